# Initial kernel scaffold; baseline (speedup 1.0000x reference)
import jax
import jax.numpy as jnp
from jax.experimental import pallas as pl


def kernel(hv, W, b, dest):
    raise NotImplementedError("write your pallas kernel here")


# speedup vs baseline: 2.4246x; 2.4246x over previous
import jax
import jax.numpy as jnp
from jax.experimental import pallas as pl
from jax.experimental.pallas import tpu as pltpu

_N = 50000
_D = 512
_S = _N - 1
_NB = 10
_BR = _N // _NB


def _body(dest_ref, hv_ref, w_ref, probs_ref, logp_ref, scores_ref):
    i = pl.program_id(0)

    @pl.when(i < _NB)
    def _phase1():
        w1 = w_ref[:, :_D]
        blk = hv_ref[...]
        chunk = jax.lax.dot_general(
            w1, blk, (((1,), (1,)), ((), ())),
            preferred_element_type=jnp.float32)
        riota = jax.lax.broadcasted_iota(jnp.int32, (_NB, _BR), 0)
        bcast = jnp.broadcast_to(chunk, (_NB, _BR))
        scores_ref[...] = jnp.where(riota == i, bcast, scores_ref[...])

    @pl.when(i == _NB)
    def _phase2():
        s = scores_ref[...]
        riota = jax.lax.broadcasted_iota(jnp.int32, (_NB, _BR), 0)
        ciota = jax.lax.broadcasted_iota(jnp.int32, (_NB, _BR), 1)
        neg_inf = jnp.float32(-jnp.inf)
        s = jnp.where((riota == _NB - 1) & (ciota == _BR - 1), neg_inf, s)
        m = jnp.max(s)
        e = jnp.exp(s - m)
        z = jnp.sum(e)
        probs_ref[...] = e * (1.0 / z)
        d = dest_ref[0]
        sel = jnp.where((riota == d // _BR) & (ciota == d % _BR), s, neg_inf)
        sd = jnp.max(sel)
        logp_ref[...] = jnp.broadcast_to(sd - m - jnp.log(z), (1, 1))


def kernel(hv, W, b, dest):
    del b
    dest_arr = jnp.asarray(dest, dtype=jnp.int32).reshape((1,))
    probs_full, logp = pl.pallas_call(
        _body,
        grid=(_NB + 1,),
        in_specs=[
            pl.BlockSpec(memory_space=pltpu.SMEM),
            pl.BlockSpec((_BR, _D), lambda i: (jnp.minimum(i, _NB - 1), 0)),
            pl.BlockSpec((1, 2 * _D), lambda i: (0, 0)),
        ],
        out_specs=[
            pl.BlockSpec((_NB, _BR), lambda i: (0, 0)),
            pl.BlockSpec((1, 1), lambda i: (0, 0)),
        ],
        out_shape=[
            jax.ShapeDtypeStruct((_NB, _BR), jnp.float32),
            jax.ShapeDtypeStruct((1, 1), jnp.float32),
        ],
        scratch_shapes=[pltpu.VMEM((_NB, _BR), jnp.float32)],
    )(dest_arr, hv, W)
    probs = probs_full.reshape(1, _N)[:, :_S]
    return (probs, logp)
